# Initial kernel scaffold; baseline (speedup 1.0000x reference)
import jax
import jax.numpy as jnp
from jax.experimental import pallas as pl


def kernel(maps, i2t_wemb, nmt_wemb):
    raise NotImplementedError("write your pallas kernel here")


# speedup vs baseline: 1.0087x; 1.0087x over previous
import functools

import jax
import jax.numpy as jnp
from jax import lax
from jax.experimental import pallas as pl
from jax.experimental.pallas import tpu as pltpu
from jax.experimental.pallas import tpu_sc as plsc

N = 100000
D = 64
CHUNK = 80
NUM_CHUNKS = N // CHUNK
NC = 2
NS = 16
NW = NC * NS
L = 16


def _sc_body(i2t_hbm, nmt_hbm, idx0_hbm, idx1_hbm, out_hbm,
             idx0_v, idx1_v, a_v, b_v, acc_v, sem0, sem1):
    wid = lax.axis_index("s") * NC + lax.axis_index("c")
    n_slots = jnp.where(wid < NUM_CHUNKS - 39 * NW, 40, 39)

    def chunk_body(c, accs):
        g = wid + c * NW
        base = g * CHUNK
        pltpu.sync_copy(idx0_hbm.at[pl.ds(base, CHUNK)], idx0_v)
        pltpu.sync_copy(idx1_hbm.at[pl.ds(base, CHUNK)], idx1_v)
        cp0 = pltpu.async_copy(i2t_hbm.at[idx0_v], a_v, sem0)
        cp1 = pltpu.async_copy(nmt_hbm.at[idx1_v], b_v, sem1)
        cp0.wait()
        cp1.wait()

        def row_body(i, accs):
            a0, a1, a2, a3 = accs
            d0 = a_v[i, pl.ds(0, L)] - b_v[i, pl.ds(0, L)]
            d1 = a_v[i, pl.ds(L, L)] - b_v[i, pl.ds(L, L)]
            d2 = a_v[i, pl.ds(2 * L, L)] - b_v[i, pl.ds(2 * L, L)]
            d3 = a_v[i, pl.ds(3 * L, L)] - b_v[i, pl.ds(3 * L, L)]
            return (a0 + d0 * d0, a1 + d1 * d1, a2 + d2 * d2, a3 + d3 * d3)

        return lax.fori_loop(0, CHUNK, row_body, accs)

    zero = jnp.zeros((L,), jnp.float32)
    accs = lax.fori_loop(0, n_slots, chunk_body, (zero, zero, zero, zero))
    total = (accs[0] + accs[1]) + (accs[2] + accs[3])
    acc_v[...] = total * jnp.float32(1.0 / (N * D))
    pltpu.sync_copy(acc_v, out_hbm.at[wid])


@jax.jit
def _sc_mse(i2t_wemb, nmt_wemb, idx0, idx1):
    mesh = plsc.VectorSubcoreMesh(core_axis_name="c", subcore_axis_name="s",
                                  num_cores=NC, num_subcores=NS)
    f = pl.kernel(
        _sc_body,
        out_type=jax.ShapeDtypeStruct((NW, L), jnp.float32),
        mesh=mesh,
        scratch_types=[
            pltpu.VMEM((CHUNK,), jnp.int32),
            pltpu.VMEM((CHUNK,), jnp.int32),
            pltpu.VMEM((CHUNK, D), jnp.float32),
            pltpu.VMEM((CHUNK, D), jnp.float32),
            pltpu.VMEM((L,), jnp.float32),
            pltpu.SemaphoreType.DMA,
            pltpu.SemaphoreType.DMA,
        ],
        compiler_params=pltpu.CompilerParams(use_tc_tiling_on_sc=False),
    )
    return f(i2t_wemb, nmt_wemb, idx0, idx1)


def kernel(maps, i2t_wemb, nmt_wemb):
    idx0 = maps[:, 0]
    idx1 = maps[:, 1]
    partials = _sc_mse(i2t_wemb, nmt_wemb, idx0, idx1)
    return jnp.sum(partials)


# speedup vs baseline: 1.0276x; 1.0187x over previous
import jax
import jax.numpy as jnp
from jax import lax
from jax.experimental import pallas as pl
from jax.experimental.pallas import tpu as pltpu
from jax.experimental.pallas import tpu_sc as plsc

N = 100000
D = 64
CHUNK = 80
VALID_CHUNKS = N // CHUNK
NC = 2
NS = 16
NW = NC * NS
CPW = 40
PAD_CHUNKS = NW * CPW
NBUF = 4
L = 16
SCALE = 1.0 / (N * D)


def _sc_body(i2t_hbm, nmt_hbm, idx0_hbm, idx1_hbm, out_hbm,
             idx0_v, idx1_v, acc_v, a_bufs, b_bufs, sems):
    wid = lax.axis_index("s") * NC + lax.axis_index("c")

    pltpu.sync_copy(idx0_hbm.at[pl.ds(wid * CPW, CPW)], idx0_v)
    pltpu.sync_copy(idx1_hbm.at[pl.ds(wid * CPW, CPW)], idx1_v)

    def issue(c, b):
        pltpu.async_copy(i2t_hbm.at[idx0_v.at[c]], a_bufs[b], sems[b])
        pltpu.async_copy(nmt_hbm.at[idx1_v.at[c]], b_bufs[b], sems[b])

    def wait(b):
        pltpu.make_async_copy(i2t_hbm.at[idx0_v.at[0]], a_bufs[b], sems[b]).wait()
        pltpu.make_async_copy(nmt_hbm.at[idx1_v.at[0]], b_bufs[b], sems[b]).wait()

    for b in range(NBUF):
        issue(b, b)

    def outer_body(k, accs):
        for b in range(NBUF):
            c = k * NBUF + b
            wait(b)
            a_v, b_v = a_bufs[b], b_bufs[b]

            def row_body(i, ch):
                out = list(ch)
                for r in range(4):
                    row = i * 4 + r
                    for j in range(4):
                        d = (a_v[row, pl.ds(j * L, L)]
                             - b_v[row, pl.ds(j * L, L)])
                        out[j] = out[j] + d * d
                return tuple(out)

            zero = jnp.zeros((L,), jnp.float32)
            ch = lax.fori_loop(0, CHUNK // 4, row_body,
                               (zero, zero, zero, zero))
            w = jnp.where(wid * CPW + c < VALID_CHUNKS,
                          jnp.float32(1.0), jnp.float32(0.0))
            accs = tuple(t + w * p for t, p in zip(accs, ch))

            @pl.when(c + NBUF < CPW)
            def _():
                issue(c + NBUF, b)
        return accs

    zero = jnp.zeros((L,), jnp.float32)
    accs = lax.fori_loop(0, CPW // NBUF, outer_body, (zero,) * 4)
    total = (accs[0] + accs[1]) + (accs[2] + accs[3])
    acc_v[...] = total * jnp.float32(SCALE)
    pltpu.sync_copy(acc_v, out_hbm.at[wid])


@jax.jit
def _sc_mse(i2t_wemb, nmt_wemb, idx0, idx1):
    mesh = plsc.VectorSubcoreMesh(core_axis_name="c", subcore_axis_name="s",
                                  num_cores=NC, num_subcores=NS)
    f = pl.kernel(
        _sc_body,
        out_type=jax.ShapeDtypeStruct((NW, L), jnp.float32),
        mesh=mesh,
        scratch_types=[
            pltpu.VMEM((CPW, CHUNK), jnp.int32),
            pltpu.VMEM((CPW, CHUNK), jnp.int32),
            pltpu.VMEM((L,), jnp.float32),
            [pltpu.VMEM((CHUNK, D), jnp.float32) for _ in range(NBUF)],
            [pltpu.VMEM((CHUNK, D), jnp.float32) for _ in range(NBUF)],
            [pltpu.SemaphoreType.DMA for _ in range(NBUF)],
        ],
        compiler_params=pltpu.CompilerParams(use_tc_tiling_on_sc=False),
    )
    return f(i2t_wemb, nmt_wemb, idx0, idx1)


def kernel(maps, i2t_wemb, nmt_wemb):
    pad = PAD_CHUNKS * CHUNK - N
    idx0 = jnp.pad(maps[:, 0], (0, pad)).reshape(PAD_CHUNKS, CHUNK)
    idx1 = jnp.pad(maps[:, 1], (0, pad)).reshape(PAD_CHUNKS, CHUNK)
    partials = _sc_mse(i2t_wemb, nmt_wemb, idx0, idx1)
    return jnp.sum(partials)


# speedup vs baseline: 1.0689x; 1.0402x over previous
import jax
import jax.numpy as jnp
from jax import lax
from jax.experimental import pallas as pl
from jax.experimental.pallas import tpu as pltpu
from jax.experimental.pallas import tpu_sc as plsc

N = 100000
D = 64
CHUNK = 80
VALID_CHUNKS = N // CHUNK
NC = 2
NS = 16
NW = NC * NS
CPW = 40
NBUF = 4
L = 16
SCALE = 1.0 / (N * D)


def _sc_body(i2t_hbm, nmt_hbm, maps_hbm, out_hbm,
             stage_v, acc_v, idx0_bufs, idx1_bufs, a_bufs, b_bufs, sems):
    wid = lax.axis_index("s") * NC + lax.axis_index("c")
    base_chunk = jnp.minimum(wid * CPW, VALID_CHUNKS - CPW)

    pltpu.sync_copy(maps_hbm.at[pl.ds(base_chunk * (CHUNK * 2), CPW * CHUNK * 2)],
                    stage_v)

    lane = lax.iota(jnp.int32, 16)

    def prep_issue(c, b):
        g = wid * CPW + c
        gc = jnp.minimum(g, VALID_CHUNKS - 1)
        off = (gc - base_chunk) * (CHUNK * 2)
        for t in range(CHUNK // L):
            base = off + t * (2 * L) + 2 * lane
            idx0_bufs[b][pl.ds(t * L, L)] = plsc.load_gather(stage_v, [base])
            idx1_bufs[b][pl.ds(t * L, L)] = plsc.load_gather(stage_v, [base + 1])
        pltpu.async_copy(i2t_hbm.at[idx0_bufs[b]], a_bufs[b], sems[b])
        pltpu.async_copy(nmt_hbm.at[idx1_bufs[b]], b_bufs[b], sems[b])

    def wait(b):
        pltpu.make_async_copy(i2t_hbm.at[idx0_bufs[b]], a_bufs[b], sems[b]).wait()
        pltpu.make_async_copy(nmt_hbm.at[idx1_bufs[b]], b_bufs[b], sems[b]).wait()

    for b in range(NBUF):
        prep_issue(b, b)

    def outer_body(k, accs):
        for b in range(NBUF):
            c = k * NBUF + b
            wait(b)
            a_v, b_v = a_bufs[b], b_bufs[b]

            def row_body(i, ch):
                out = list(ch)
                for r in range(4):
                    row = i * 4 + r
                    for j in range(4):
                        d = (a_v[row, pl.ds(j * L, L)]
                             - b_v[row, pl.ds(j * L, L)])
                        out[j] = out[j] + d * d
                return tuple(out)

            zero = jnp.zeros((L,), jnp.float32)
            ch = lax.fori_loop(0, CHUNK // 4, row_body,
                               (zero, zero, zero, zero))
            w = jnp.where(wid * CPW + c < VALID_CHUNKS,
                          jnp.float32(1.0), jnp.float32(0.0))
            accs = tuple(t + w * p for t, p in zip(accs, ch))

            @pl.when(c + NBUF < CPW)
            def _():
                prep_issue(c + NBUF, b)
        return accs

    zero = jnp.zeros((L,), jnp.float32)
    accs = lax.fori_loop(0, CPW // NBUF, outer_body, (zero,) * 4)
    total = (accs[0] + accs[1]) + (accs[2] + accs[3])
    acc_v[...] = total * jnp.float32(SCALE)
    pltpu.sync_copy(acc_v, out_hbm.at[wid])


@jax.jit
def _sc_mse(i2t_wemb, nmt_wemb, maps_flat):
    mesh = plsc.VectorSubcoreMesh(core_axis_name="c", subcore_axis_name="s",
                                  num_cores=NC, num_subcores=NS)
    f = pl.kernel(
        _sc_body,
        out_type=jax.ShapeDtypeStruct((NW, L), jnp.float32),
        mesh=mesh,
        scratch_types=[
            pltpu.VMEM((CPW * CHUNK * 2,), jnp.int32),
            pltpu.VMEM((L,), jnp.float32),
            [pltpu.VMEM((CHUNK,), jnp.int32) for _ in range(NBUF)],
            [pltpu.VMEM((CHUNK,), jnp.int32) for _ in range(NBUF)],
            [pltpu.VMEM((CHUNK, D), jnp.float32) for _ in range(NBUF)],
            [pltpu.VMEM((CHUNK, D), jnp.float32) for _ in range(NBUF)],
            [pltpu.SemaphoreType.DMA for _ in range(NBUF)],
        ],
        compiler_params=pltpu.CompilerParams(use_tc_tiling_on_sc=False,
                                             needs_layout_passes=False),
    )
    return f(i2t_wemb, nmt_wemb, maps_flat)


def kernel(maps, i2t_wemb, nmt_wemb):
    partials = _sc_mse(i2t_wemb, nmt_wemb, maps.reshape(-1))
    return jnp.sum(partials)
